# baseline (speedup 1.0000x reference)
import dataclasses
import functools

import jax
import jax.numpy as jnp
from jax import lax
from jax.experimental import pallas as pl
from jax.experimental.pallas import tpu as pltpu
from jax.experimental.pallas import tpu_sc as plsc

H = 512
W = 512
H2 = 1024
W2 = 1024
NSENT = H2 * W2

NWORK = 32
BAND = 16
CH = 4096
SCH = 8192
PREP_NC = 8192


def _prep_body(pc_ref, conf_ref, par_ref, sloc_ref, nbr_ref, wts_ref):
    bi = pl.program_id(0)
    px = pc_ref[0, 0:1, :]
    py = pc_ref[0, 1:2, :]
    pz = pc_ref[0, 2:3, :]
    cf = conf_ref[0, 0:1, :]
    k00 = par_ref[0, 0, 0]
    k02 = par_ref[0, 0, 1]
    k11 = par_ref[0, 0, 2]
    k12 = par_ref[0, 0, 3]

    absz = jnp.abs(pz)
    xc = px * k00 / absz + k02
    yc = py * k11 / absz + k12

    x_norm = xc / float(W - 1) * 2.0 - 1.0
    y_norm = -(yc / float(H - 1) * 2.0 - 1.0)
    ix = (x_norm + 1.0) * 0.5 * (W - 1)
    iy = (y_norm + 1.0) * 0.5 * (H - 1)
    x0 = jnp.floor(ix)
    y0 = jnp.floor(iy)
    x1 = x0 + 1.0
    y1 = y0 + 1.0
    wx1 = ix - x0
    wx0 = 1.0 - wx1
    wy1 = iy - y0
    wy0 = 1.0 - wy1
    j = pl.program_id(1)
    spread = (lax.broadcasted_iota(jnp.int32, (1, PREP_NC), 1)
              + j * PREP_NC) & (H * W - 1)

    def nbr(xi, yi, wgt):
        valid = (xi >= 0) & (xi <= W - 1) & (yi >= 0) & (yi <= H - 1)
        xcl = jnp.clip(xi, 0, W - 1).astype(jnp.int32)
        ycl = jnp.clip(yi, 0, H - 1).astype(jnp.int32)
        idx = jnp.where(valid, ycl * W + xcl, spread)
        return idx, wgt * valid.astype(jnp.float32)

    i00, w00 = nbr(x0, y0, wy0 * wx0)
    i01, w01 = nbr(x1, y0, wy0 * wx1)
    i10, w10 = nbr(x0, y1, wy1 * wx0)
    i11, w11 = nbr(x1, y1, wy1 * wx1)
    il0 = jnp.stack([i00, i01], axis=-1).reshape(1, 2 * PREP_NC)
    il1 = jnp.stack([i10, i11], axis=-1).reshape(1, 2 * PREP_NC)
    nbr_ref[0] = jnp.concatenate([il0, il1], axis=0)
    wts_ref[0] = jnp.concatenate([w00, w01, w10, w11], axis=0)

    xch = xc * 2
    ych = yc * 2
    xr = jnp.round(xch)
    yr = jnp.round(ych)
    oob = ((xr < 0) | (xr >= W2) | (yr < 0) | (yr >= H2)
           | (absz < 0.1) | (absz > 100.0) | (cf <= 0))
    valid = jnp.logical_not(oob)
    xi = jnp.clip(xr, 0, W2 - 1).astype(jnp.int32)
    yi = jnp.clip(yr, 0, H2 - 1).astype(jnp.int32)
    p = ((yi & 1) << 1) | (xi & 1)
    pr = (H - 1) - (yi >> 1)
    pcol = xi >> 1
    sloc = (pr * 4 + p) * W + pcol
    sloc_ref[0] = jnp.where(valid, sloc, NSENT)


def _sc_scatter_body(sloc_hbm, conf_hbm, cout_hbm, iout_hbm,
                     cbuf, ibuf, slbuf, cfbuf, slbuf2, cfbuf2, sem, sem2):
    wid = lax.axis_index("c") * 16 + lax.axis_index("s")
    r0 = wid * BAND
    lo = r0 * (4 * W)
    hi = lo + BAND * 4 * W
    zf = jnp.zeros((16,), jnp.float32)
    zneg = jnp.full((16,), -1, jnp.int32)
    lanes = lax.iota(jnp.int32, 16)

    @pl.loop(0, 4)
    def _batch(bi):
        pltpu.async_copy(sloc_hbm.at[bi, pl.ds(0, SCH)], slbuf, sem)
        pltpu.async_copy(conf_hbm.at[bi, pl.ds(0, SCH)], cfbuf, sem)

        @pl.loop(0, 4)
        def _mp(p_):
            @pl.loop(0, BAND)
            def _mr(r_):
                @pl.loop(0, W // 16)
                def _mc(c_):
                    cbuf[p_, r_, pl.ds(c_ * 16, 16)] = zf
                    ibuf[p_, r_, pl.ds(c_ * 16, 16)] = zneg

        def _do_chunk(base, slb, cfb):
            @pl.loop(0, SCH // 16)
            def _vec(v):
                sl = slb[pl.ds(v * 16, 16)]
                cf = cfb[pl.ds(v * 16, 16)]
                m = (sl >= lo) & (sl < hi)
                idxv = base + v * 16 + lanes
                sls = jnp.where(m, sl - lo, 0)
                pv = (sls >> 9) & 3
                lr = sls >> 11
                pc_ = sls & 511
                idxs3 = [pv, lr, pc_]
                plsc.store_scatter(ibuf, idxs3, idxv, mask=m)
                plsc.store_scatter(cbuf, idxs3, cf, mask=m)

        @pl.loop(0, 131072 // (2 * SCH))
        def _chunk(c):
            base = c * 2 * SCH
            pltpu.async_copy(sloc_hbm.at[bi, pl.ds(base + SCH, SCH)],
                             slbuf2, sem2)
            pltpu.async_copy(conf_hbm.at[bi, pl.ds(base + SCH, SCH)],
                             cfbuf2, sem2)
            pltpu.make_async_copy(sloc_hbm.at[bi, pl.ds(0, SCH)], slbuf,
                                  sem).wait()
            pltpu.make_async_copy(conf_hbm.at[bi, pl.ds(0, SCH)], cfbuf,
                                  sem).wait()
            _do_chunk(base, slbuf, cfbuf)
            nxt = base + 2 * SCH

            @pl.when(nxt < 131072)
            def _():
                pltpu.async_copy(sloc_hbm.at[bi, pl.ds(nxt, SCH)], slbuf, sem)
                pltpu.async_copy(conf_hbm.at[bi, pl.ds(nxt, SCH)], cfbuf, sem)

            pltpu.make_async_copy(sloc_hbm.at[bi, pl.ds(0, SCH)], slbuf2,
                                  sem2).wait()
            pltpu.make_async_copy(conf_hbm.at[bi, pl.ds(0, SCH)], cfbuf2,
                                  sem2).wait()
            _do_chunk(base + SCH, slbuf2, cfbuf2)

        for p_ in range(4):
            pltpu.sync_copy(cbuf.at[p_], cout_hbm.at[bi, p_, pl.ds(r0, BAND)])
            pltpu.sync_copy(ibuf.at[p_], iout_hbm.at[bi, p_, pl.ds(r0, BAND)])


def _sc_gather_body(beta_hbm, nbr_hbm, wts_hbm, out_hbm,
                    i0, i1, w0, w1, w2, w3, g0, g1,
                    obuf, bstage, sem):
    sid = lax.axis_index("s")
    wid = lax.axis_index("c") * 16 + sid
    base = wid * CH
    ibufs = [i0, i1]
    wbufs = [w0, w1, w2, w3]
    gbufs = [g0, g1]
    lanes2 = lax.iota(jnp.int32, 16) * 2

    @pl.loop(0, 4)
    def _batch(bi):
        plsc.subcore_barrier()

        @pl.when(sid == 0)
        def _():
            pltpu.sync_copy(beta_hbm.at[bi], bstage)

        plsc.subcore_barrier()
        for j in range(2):
            pltpu.sync_copy(nbr_hbm.at[bi, j, pl.ds(2 * base, 2 * CH)],
                            ibufs[j])
        for j in range(4):
            pltpu.sync_copy(wts_hbm.at[bi, j, pl.ds(base, CH)], wbufs[j])
        cps = [pltpu.async_copy(bstage.at[ibufs[j]], gbufs[j], sem)
               for j in range(2)]
        for cp in cps:
            cp.wait()

        @pl.loop(0, CH // 16)
        def _vec(v):
            s = pl.ds(v * 16, 16)
            ie = v * 32 + lanes2
            io = ie + 1
            g00 = plsc.load_gather(g0, [ie])
            g01 = plsc.load_gather(g0, [io])
            g10 = plsc.load_gather(g1, [ie])
            g11 = plsc.load_gather(g1, [io])
            acc = ((w0[s] * g00 + w1[s] * g01)
                   + w2[s] * g10) + w3[s] * g11
            obuf[s] = acc

        pltpu.sync_copy(obuf, out_hbm.at[bi, pl.ds(base, CH)])


def _pool_body(cpl_ref, ipl_ref, alpha_ref, match_ref, src_ref):
    a = alpha_ref[0, 0]
    A = a > 0.0
    bv = None
    for p_ in range(4):
        Cp = cpl_ref[0, p_]
        Ip = ipl_ref[0, p_]
        k = A & (Cp > 0.0)
        v = jnp.where(k, Cp, 0.0)
        im = jnp.where(k, Ip, -1)
        if p_ == 0:
            bv, bi_, anyk = v, im, k
        else:
            upd = v > bv
            bv = jnp.where(upd, v, bv)
            bi_ = jnp.where(upd, im, bi_)
            anyk = anyk | k
    match_ref[0, 0] = anyk.astype(jnp.float32)
    src_ref[0, 0] = bi_


def kernel(alpha, beta, pp, conf, pose_w2c, K, h, w):
    b = pp.shape[0]
    n = pp.shape[2]
    h, w = alpha.shape[-2], alpha.shape[-1]
    f32 = jnp.float32
    i32 = jnp.int32

    pc = jnp.einsum('bij,bjn->bin', pose_w2c, pp[:, :4, :])
    par = jnp.stack([K[:, 0, 0], K[:, 0, 2], K[:, 1, 1], K[:, 1, 2]],
                    axis=1).reshape(b, 1, 4)
    conf3 = conf.reshape(b, 1, n)

    grid = (b, n // PREP_NC)
    sloc3, nbr, wts = pl.pallas_call(
        _prep_body,
        grid=grid,
        in_specs=[
            pl.BlockSpec((1, 4, PREP_NC), lambda bi, j: (bi, 0, j)),
            pl.BlockSpec((1, 1, PREP_NC), lambda bi, j: (bi, 0, j)),
            pl.BlockSpec((1, 1, 4), lambda bi, j: (bi, 0, 0)),
        ],
        out_specs=[
            pl.BlockSpec((1, 1, PREP_NC), lambda bi, j: (bi, 0, j)),
            pl.BlockSpec((1, 2, 2 * PREP_NC), lambda bi, j: (bi, 0, j)),
            pl.BlockSpec((1, 4, PREP_NC), lambda bi, j: (bi, 0, j)),
        ],
        out_shape=[
            jax.ShapeDtypeStruct((b, 1, n), i32),
            jax.ShapeDtypeStruct((b, 2, 2 * n), i32),
            jax.ShapeDtypeStruct((b, 4, n), f32),
        ],
    )(pc, conf3, par)
    sloc = sloc3.reshape(b, n)

    mesh = plsc.VectorSubcoreMesh(core_axis_name="c", subcore_axis_name="s",
                                  num_cores=2, num_subcores=16)
    sc_cp = pltpu.CompilerParams()
    if "needs_layout_passes" in pltpu.CompilerParams.__dataclass_fields__:
        sc_cp = dataclasses.replace(sc_cp, needs_layout_passes=False)

    sc_scatter = pl.kernel(
        _sc_scatter_body,
        out_type=[jax.ShapeDtypeStruct((b, 4, H, W), f32),
                  jax.ShapeDtypeStruct((b, 4, H, W), i32)],
        mesh=mesh,
        scratch_types=[
            pltpu.VMEM((4, BAND, W), f32),
            pltpu.VMEM((4, BAND, W), i32),
            pltpu.VMEM((SCH,), i32),
            pltpu.VMEM((SCH,), f32),
            pltpu.VMEM((SCH,), i32),
            pltpu.VMEM((SCH,), f32),
            pltpu.SemaphoreType.DMA,
            pltpu.SemaphoreType.DMA,
        ],
        compiler_params=sc_cp,
    )
    cplanes, iplanes = sc_scatter(sloc, conf)

    sc_gather = pl.kernel(
        _sc_gather_body,
        out_type=jax.ShapeDtypeStruct((b, n), f32),
        mesh=mesh,
        scratch_types=(
            [pltpu.VMEM((2 * CH,), i32)] * 2
            + [pltpu.VMEM((CH,), f32)] * 4
            + [pltpu.VMEM((2 * CH,), f32)] * 2
            + [pltpu.VMEM((CH,), f32),
               pltpu.VMEM_SHARED((H * W,), f32),
               pltpu.SemaphoreType.DMA]
        ),
        compiler_params=sc_cp,
    )
    conf_sampled = sc_gather(beta.reshape(b, h * w), nbr, wts)

    R = 256
    match, src = pl.pallas_call(
        _pool_body,
        grid=(b, H // R),
        in_specs=[
            pl.BlockSpec((1, 4, R, W), lambda bi, rj: (bi, 0, rj, 0)),
            pl.BlockSpec((1, 4, R, W), lambda bi, rj: (bi, 0, rj, 0)),
            pl.BlockSpec((1, 1, R, W), lambda bi, rj: (bi, 0, rj, 0)),
        ],
        out_specs=[
            pl.BlockSpec((1, 1, R, W), lambda bi, rj: (bi, 0, rj, 0)),
            pl.BlockSpec((1, 1, R, W), lambda bi, rj: (bi, 0, rj, 0)),
        ],
        out_shape=[
            jax.ShapeDtypeStruct((b, 1, H, W), f32),
            jax.ShapeDtypeStruct((b, 1, H, W), i32),
        ],
    )(cplanes, iplanes, alpha)

    return (match, src, conf_sampled.reshape(b, 1, n))


# speedup vs baseline: 2.9234x; 2.9234x over previous
import dataclasses
import functools

import jax
import jax.numpy as jnp
from jax import lax
from jax.experimental import pallas as pl
from jax.experimental.pallas import tpu as pltpu
from jax.experimental.pallas import tpu_sc as plsc

H = 512
W = 512
H2 = 1024
W2 = 1024
NSENT = H2 * W2

NWORK = 32
BAND = 16
CH = 4096
SCH = 8192
PREP_NC = 8192


def _prep_body(pc_ref, conf_ref, par_ref, sloc_ref, nbr_ref, wts_ref):
    bi = pl.program_id(0)
    px = pc_ref[0, 0:1, :]
    py = pc_ref[0, 1:2, :]
    pz = pc_ref[0, 2:3, :]
    cf = conf_ref[0, 0:1, :]
    k00 = par_ref[0, 0, 0]
    k02 = par_ref[0, 0, 1]
    k11 = par_ref[0, 0, 2]
    k12 = par_ref[0, 0, 3]

    absz = jnp.abs(pz)
    xc = px * k00 / absz + k02
    yc = py * k11 / absz + k12

    x_norm = xc / float(W - 1) * 2.0 - 1.0
    y_norm = -(yc / float(H - 1) * 2.0 - 1.0)
    ix = (x_norm + 1.0) * 0.5 * (W - 1)
    iy = (y_norm + 1.0) * 0.5 * (H - 1)
    x0 = jnp.floor(ix)
    y0 = jnp.floor(iy)
    x1 = x0 + 1.0
    y1 = y0 + 1.0
    wx1 = ix - x0
    wx0 = 1.0 - wx1
    wy1 = iy - y0
    wy0 = 1.0 - wy1
    j = pl.program_id(1)
    spread = (lax.broadcasted_iota(jnp.int32, (1, PREP_NC), 1)
              + j * PREP_NC) & (H * W - 1)

    def nbr(xi, yi, wgt):
        valid = (xi >= 0) & (xi <= W - 1) & (yi >= 0) & (yi <= H - 1)
        xcl = jnp.clip(xi, 0, W - 1).astype(jnp.int32)
        ycl = jnp.clip(yi, 0, H - 1).astype(jnp.int32)
        idx = jnp.where(valid, ycl * W + xcl, spread)
        return idx, wgt * valid.astype(jnp.float32)

    i00, w00 = nbr(x0, y0, wy0 * wx0)
    i01, w01 = nbr(x1, y0, wy0 * wx1)
    i10, w10 = nbr(x0, y1, wy1 * wx0)
    i11, w11 = nbr(x1, y1, wy1 * wx1)
    nbr_ref[0] = jnp.concatenate([i00, i01, i10, i11], axis=0)
    wts_ref[0] = jnp.concatenate([w00, w01, w10, w11], axis=0)

    xch = xc * 2
    ych = yc * 2
    xr = jnp.round(xch)
    yr = jnp.round(ych)
    oob = ((xr < 0) | (xr >= W2) | (yr < 0) | (yr >= H2)
           | (absz < 0.1) | (absz > 100.0) | (cf <= 0))
    valid = jnp.logical_not(oob)
    xi = jnp.clip(xr, 0, W2 - 1).astype(jnp.int32)
    yi = jnp.clip(yr, 0, H2 - 1).astype(jnp.int32)
    p = ((yi & 1) << 1) | (xi & 1)
    pr = (H - 1) - (yi >> 1)
    pcol = xi >> 1
    sloc = (pr * 4 + p) * W + pcol
    sloc_ref[0] = jnp.where(valid, sloc, NSENT)


def _sc_scatter_body(sloc_hbm, conf_hbm, cout_hbm, iout_hbm,
                     cbuf, ibuf, slbuf, cfbuf, slbuf2, cfbuf2, sem, sem2):
    wid = lax.axis_index("c") * 16 + lax.axis_index("s")
    r0 = wid * BAND
    lo = r0 * (4 * W)
    hi = lo + BAND * 4 * W
    zf = jnp.zeros((16,), jnp.float32)
    zneg = jnp.full((16,), -1, jnp.int32)
    lanes = lax.iota(jnp.int32, 16)

    @pl.loop(0, 4)
    def _batch(bi):
        pltpu.async_copy(sloc_hbm.at[bi, pl.ds(0, SCH)], slbuf, sem)
        pltpu.async_copy(conf_hbm.at[bi, pl.ds(0, SCH)], cfbuf, sem)

        @pl.loop(0, 4)
        def _mp(p_):
            @pl.loop(0, BAND)
            def _mr(r_):
                @pl.loop(0, W // 16)
                def _mc(c_):
                    cbuf[p_, r_, pl.ds(c_ * 16, 16)] = zf
                    ibuf[p_, r_, pl.ds(c_ * 16, 16)] = zneg

        def _do_chunk(base, slb, cfb):
            @pl.loop(0, SCH // 16)
            def _vec(v):
                sl = slb[pl.ds(v * 16, 16)]
                cf = cfb[pl.ds(v * 16, 16)]
                m = (sl >= lo) & (sl < hi)
                idxv = base + v * 16 + lanes
                sls = jnp.where(m, sl - lo, 0)
                pv = (sls >> 9) & 3
                lr = sls >> 11
                pc_ = sls & 511
                idxs3 = [pv, lr, pc_]
                plsc.store_scatter(ibuf, idxs3, idxv, mask=m)
                plsc.store_scatter(cbuf, idxs3, cf, mask=m)

        @pl.loop(0, 131072 // (2 * SCH))
        def _chunk(c):
            base = c * 2 * SCH
            pltpu.async_copy(sloc_hbm.at[bi, pl.ds(base + SCH, SCH)],
                             slbuf2, sem2)
            pltpu.async_copy(conf_hbm.at[bi, pl.ds(base + SCH, SCH)],
                             cfbuf2, sem2)
            pltpu.make_async_copy(sloc_hbm.at[bi, pl.ds(0, SCH)], slbuf,
                                  sem).wait()
            pltpu.make_async_copy(conf_hbm.at[bi, pl.ds(0, SCH)], cfbuf,
                                  sem).wait()
            _do_chunk(base, slbuf, cfbuf)
            nxt = base + 2 * SCH

            @pl.when(nxt < 131072)
            def _():
                pltpu.async_copy(sloc_hbm.at[bi, pl.ds(nxt, SCH)], slbuf, sem)
                pltpu.async_copy(conf_hbm.at[bi, pl.ds(nxt, SCH)], cfbuf, sem)

            pltpu.make_async_copy(sloc_hbm.at[bi, pl.ds(0, SCH)], slbuf2,
                                  sem2).wait()
            pltpu.make_async_copy(conf_hbm.at[bi, pl.ds(0, SCH)], cfbuf2,
                                  sem2).wait()
            _do_chunk(base + SCH, slbuf2, cfbuf2)

        for p_ in range(4):
            pltpu.sync_copy(cbuf.at[p_], cout_hbm.at[bi, p_, pl.ds(r0, BAND)])
            pltpu.sync_copy(ibuf.at[p_], iout_hbm.at[bi, p_, pl.ds(r0, BAND)])


def _sc_gather_body(beta_hbm, nbr_hbm, wts_hbm, out_hbm,
                    i0, i1, i2, i3, w0, w1, w2, w3, g0, g1, g2, g3,
                    obuf, bstage, sem):
    sid = lax.axis_index("s")
    wid = lax.axis_index("c") * 16 + sid
    base = wid * CH
    ibufs = [i0, i1, i2, i3]
    wbufs = [w0, w1, w2, w3]
    gbufs = [g0, g1, g2, g3]

    @pl.loop(0, 4)
    def _batch(bi):
        plsc.subcore_barrier()

        @pl.when(sid == 0)
        def _():
            pltpu.sync_copy(beta_hbm.at[bi], bstage)

        plsc.subcore_barrier()
        for j in range(4):
            pltpu.sync_copy(nbr_hbm.at[bi, j, pl.ds(base, CH)], ibufs[j])
            pltpu.sync_copy(wts_hbm.at[bi, j, pl.ds(base, CH)], wbufs[j])
        cps = [pltpu.async_copy(bstage.at[ibufs[j]], gbufs[j], sem)
               for j in range(4)]
        for cp in cps:
            cp.wait()

        @pl.loop(0, CH // 16)
        def _vec(v):
            s = pl.ds(v * 16, 16)
            acc = ((w0[s] * g0[s] + w1[s] * g1[s])
                   + w2[s] * g2[s]) + w3[s] * g3[s]
            obuf[s] = acc

        pltpu.sync_copy(obuf, out_hbm.at[bi, pl.ds(base, CH)])


def _pool_body(cpl_ref, ipl_ref, alpha_ref, match_ref, src_ref):
    a = alpha_ref[0, 0]
    A = a > 0.0
    bv = None
    for p_ in range(4):
        Cp = cpl_ref[0, p_]
        Ip = ipl_ref[0, p_]
        k = A & (Cp > 0.0)
        v = jnp.where(k, Cp, 0.0)
        im = jnp.where(k, Ip, -1)
        if p_ == 0:
            bv, bi_, anyk = v, im, k
        else:
            upd = v > bv
            bv = jnp.where(upd, v, bv)
            bi_ = jnp.where(upd, im, bi_)
            anyk = anyk | k
    match_ref[0, 0] = anyk.astype(jnp.float32)
    src_ref[0, 0] = bi_


def kernel(alpha, beta, pp, conf, pose_w2c, K, h, w):
    b = pp.shape[0]
    n = pp.shape[2]
    h, w = alpha.shape[-2], alpha.shape[-1]
    f32 = jnp.float32
    i32 = jnp.int32

    pc = jnp.einsum('bij,bjn->bin', pose_w2c, pp[:, :4, :])
    par = jnp.stack([K[:, 0, 0], K[:, 0, 2], K[:, 1, 1], K[:, 1, 2]],
                    axis=1).reshape(b, 1, 4)
    conf3 = conf.reshape(b, 1, n)

    grid = (b, n // PREP_NC)
    sloc3, nbr, wts = pl.pallas_call(
        _prep_body,
        grid=grid,
        in_specs=[
            pl.BlockSpec((1, 4, PREP_NC), lambda bi, j: (bi, 0, j)),
            pl.BlockSpec((1, 1, PREP_NC), lambda bi, j: (bi, 0, j)),
            pl.BlockSpec((1, 1, 4), lambda bi, j: (bi, 0, 0)),
        ],
        out_specs=[
            pl.BlockSpec((1, 1, PREP_NC), lambda bi, j: (bi, 0, j)),
            pl.BlockSpec((1, 4, PREP_NC), lambda bi, j: (bi, 0, j)),
            pl.BlockSpec((1, 4, PREP_NC), lambda bi, j: (bi, 0, j)),
        ],
        out_shape=[
            jax.ShapeDtypeStruct((b, 1, n), i32),
            jax.ShapeDtypeStruct((b, 4, n), i32),
            jax.ShapeDtypeStruct((b, 4, n), f32),
        ],
    )(pc, conf3, par)
    sloc = sloc3.reshape(b, n)

    mesh = plsc.VectorSubcoreMesh(core_axis_name="c", subcore_axis_name="s",
                                  num_cores=2, num_subcores=16)
    sc_cp = pltpu.CompilerParams()
    if "needs_layout_passes" in pltpu.CompilerParams.__dataclass_fields__:
        sc_cp = dataclasses.replace(sc_cp, needs_layout_passes=False)

    sc_scatter = pl.kernel(
        _sc_scatter_body,
        out_type=[jax.ShapeDtypeStruct((b, 4, H, W), f32),
                  jax.ShapeDtypeStruct((b, 4, H, W), i32)],
        mesh=mesh,
        scratch_types=[
            pltpu.VMEM((4, BAND, W), f32),
            pltpu.VMEM((4, BAND, W), i32),
            pltpu.VMEM((SCH,), i32),
            pltpu.VMEM((SCH,), f32),
            pltpu.VMEM((SCH,), i32),
            pltpu.VMEM((SCH,), f32),
            pltpu.SemaphoreType.DMA,
            pltpu.SemaphoreType.DMA,
        ],
        compiler_params=sc_cp,
    )
    cplanes, iplanes = sc_scatter(sloc, conf)

    sc_gather = pl.kernel(
        _sc_gather_body,
        out_type=jax.ShapeDtypeStruct((b, n), f32),
        mesh=mesh,
        scratch_types=(
            [pltpu.VMEM((CH,), i32)] * 4
            + [pltpu.VMEM((CH,), f32)] * 4
            + [pltpu.VMEM((CH,), f32)] * 4
            + [pltpu.VMEM((CH,), f32),
               pltpu.VMEM_SHARED((H * W,), f32),
               pltpu.SemaphoreType.DMA]
        ),
        compiler_params=sc_cp,
    )
    conf_sampled = sc_gather(beta.reshape(b, h * w), nbr, wts)

    R = 256
    match, src = pl.pallas_call(
        _pool_body,
        grid=(b, H // R),
        in_specs=[
            pl.BlockSpec((1, 4, R, W), lambda bi, rj: (bi, 0, rj, 0)),
            pl.BlockSpec((1, 4, R, W), lambda bi, rj: (bi, 0, rj, 0)),
            pl.BlockSpec((1, 1, R, W), lambda bi, rj: (bi, 0, rj, 0)),
        ],
        out_specs=[
            pl.BlockSpec((1, 1, R, W), lambda bi, rj: (bi, 0, rj, 0)),
            pl.BlockSpec((1, 1, R, W), lambda bi, rj: (bi, 0, rj, 0)),
        ],
        out_shape=[
            jax.ShapeDtypeStruct((b, 1, H, W), f32),
            jax.ShapeDtypeStruct((b, 1, H, W), i32),
        ],
    )(cplanes, iplanes, alpha)

    return (match, src, conf_sampled.reshape(b, 1, n))


# speedup vs baseline: 3.0251x; 1.0348x over previous
import dataclasses
import functools

import jax
import jax.numpy as jnp
from jax import lax
from jax.experimental import pallas as pl
from jax.experimental.pallas import tpu as pltpu
from jax.experimental.pallas import tpu_sc as plsc

H = 512
W = 512
H2 = 1024
W2 = 1024
NSENT = H2 * W2

NWORK = 32
BAND = 16
CH = 4096
SCH = 8192
PREP_NC = 8192


def _prep_body(pc_ref, conf_ref, par_ref, sloc_ref, nbr_ref, wts_ref):
    bi = pl.program_id(0)
    px = pc_ref[0, 0:1, :]
    py = pc_ref[0, 1:2, :]
    pz = pc_ref[0, 2:3, :]
    cf = conf_ref[0, 0:1, :]
    k00 = par_ref[0, 0, 0]
    k02 = par_ref[0, 0, 1]
    k11 = par_ref[0, 0, 2]
    k12 = par_ref[0, 0, 3]

    absz = jnp.abs(pz)
    xc = px * k00 / absz + k02
    yc = py * k11 / absz + k12

    x_norm = xc / float(W - 1) * 2.0 - 1.0
    y_norm = -(yc / float(H - 1) * 2.0 - 1.0)
    ix = (x_norm + 1.0) * 0.5 * (W - 1)
    iy = (y_norm + 1.0) * 0.5 * (H - 1)
    x0 = jnp.floor(ix)
    y0 = jnp.floor(iy)
    x1 = x0 + 1.0
    y1 = y0 + 1.0
    wx1 = ix - x0
    wx0 = 1.0 - wx1
    wy1 = iy - y0
    wy0 = 1.0 - wy1
    j = pl.program_id(1)
    spread = (lax.broadcasted_iota(jnp.int32, (1, PREP_NC), 1)
              + j * PREP_NC) & (H * W - 1)

    def nbr(xi, yi, wgt):
        valid = (xi >= 0) & (xi <= W - 1) & (yi >= 0) & (yi <= H - 1)
        xcl = jnp.clip(xi, 0, W - 1).astype(jnp.int32)
        ycl = jnp.clip(yi, 0, H - 1).astype(jnp.int32)
        idx = jnp.where(valid, ycl * W + xcl, spread) + bi * (H * W)
        return idx, wgt * valid.astype(jnp.float32)

    i00, w00 = nbr(x0, y0, wy0 * wx0)
    i01, w01 = nbr(x1, y0, wy0 * wx1)
    i10, w10 = nbr(x0, y1, wy1 * wx0)
    i11, w11 = nbr(x1, y1, wy1 * wx1)
    nbr_ref[0] = jnp.concatenate([i00, i01, i10, i11], axis=0)
    wts_ref[0] = jnp.concatenate([w00, w01, w10, w11], axis=0)

    xch = xc * 2
    ych = yc * 2
    xr = jnp.round(xch)
    yr = jnp.round(ych)
    oob = ((xr < 0) | (xr >= W2) | (yr < 0) | (yr >= H2)
           | (absz < 0.1) | (absz > 100.0) | (cf <= 0))
    valid = jnp.logical_not(oob)
    xi = jnp.clip(xr, 0, W2 - 1).astype(jnp.int32)
    yi = jnp.clip(yr, 0, H2 - 1).astype(jnp.int32)
    p = ((yi & 1) << 1) | (xi & 1)
    pr = (H - 1) - (yi >> 1)
    pcol = xi >> 1
    sloc = (pr * 4 + p) * W + pcol
    sloc_ref[0] = jnp.where(valid, sloc, NSENT)


def _sc_scatter_body(sloc_hbm, conf_hbm, cout_hbm, iout_hbm,
                     cbuf, ibuf, slbuf, cfbuf, slbuf2, cfbuf2, sem, sem2):
    wid = lax.axis_index("c") * 16 + lax.axis_index("s")
    r0 = wid * BAND
    lo = r0 * (4 * W)
    hi = lo + BAND * 4 * W
    zf = jnp.zeros((16,), jnp.float32)
    zneg = jnp.full((16,), -1, jnp.int32)
    lanes = lax.iota(jnp.int32, 16)

    @pl.loop(0, 4)
    def _batch(bi):
        pltpu.async_copy(sloc_hbm.at[bi, pl.ds(0, SCH)], slbuf, sem)
        pltpu.async_copy(conf_hbm.at[bi, pl.ds(0, SCH)], cfbuf, sem)

        @pl.loop(0, 4)
        def _mp(p_):
            @pl.loop(0, BAND)
            def _mr(r_):
                @pl.loop(0, W // 16)
                def _mc(c_):
                    cbuf[p_, r_, pl.ds(c_ * 16, 16)] = zf
                    ibuf[p_, r_, pl.ds(c_ * 16, 16)] = zneg

        def _do_chunk(base, slb, cfb):
            @pl.loop(0, SCH // 16)
            def _vec(v):
                sl = slb[pl.ds(v * 16, 16)]
                cf = cfb[pl.ds(v * 16, 16)]
                m = (sl >= lo) & (sl < hi)
                idxv = base + v * 16 + lanes
                sls = jnp.where(m, sl - lo, 0)
                pv = (sls >> 9) & 3
                lr = sls >> 11
                pc_ = sls & 511
                idxs3 = [pv, lr, pc_]
                plsc.store_scatter(ibuf, idxs3, idxv, mask=m)
                plsc.store_scatter(cbuf, idxs3, cf, mask=m)

        @pl.loop(0, 131072 // (2 * SCH))
        def _chunk(c):
            base = c * 2 * SCH
            pltpu.async_copy(sloc_hbm.at[bi, pl.ds(base + SCH, SCH)],
                             slbuf2, sem2)
            pltpu.async_copy(conf_hbm.at[bi, pl.ds(base + SCH, SCH)],
                             cfbuf2, sem2)
            pltpu.make_async_copy(sloc_hbm.at[bi, pl.ds(0, SCH)], slbuf,
                                  sem).wait()
            pltpu.make_async_copy(conf_hbm.at[bi, pl.ds(0, SCH)], cfbuf,
                                  sem).wait()
            _do_chunk(base, slbuf, cfbuf)
            nxt = base + 2 * SCH

            @pl.when(nxt < 131072)
            def _():
                pltpu.async_copy(sloc_hbm.at[bi, pl.ds(nxt, SCH)], slbuf, sem)
                pltpu.async_copy(conf_hbm.at[bi, pl.ds(nxt, SCH)], cfbuf, sem)

            pltpu.make_async_copy(sloc_hbm.at[bi, pl.ds(0, SCH)], slbuf2,
                                  sem2).wait()
            pltpu.make_async_copy(conf_hbm.at[bi, pl.ds(0, SCH)], cfbuf2,
                                  sem2).wait()
            _do_chunk(base + SCH, slbuf2, cfbuf2)

        for p_ in range(4):
            pltpu.sync_copy(cbuf.at[p_], cout_hbm.at[bi, p_, pl.ds(r0, BAND)])
            pltpu.sync_copy(ibuf.at[p_], iout_hbm.at[bi, p_, pl.ds(r0, BAND)])


def _sc_gather_body(beta_hbm, nbr_hbm, wts_hbm, out_hbm,
                    i0, i1, i2, i3, w0, w1, w2, w3, g0, g1, g2, g3,
                    obuf, bstage, sem, gsem):
    sid = lax.axis_index("s")
    wid = lax.axis_index("c") * 16 + sid
    base = wid * CH
    ibufs = [i0, i1, i2, i3]
    wbufs = [w0, w1, w2, w3]
    gbufs = [g0, g1, g2, g3]

    @pl.when(sid < 4)
    def _():
        pltpu.sync_copy(beta_hbm.at[sid],
                        bstage.at[pl.ds(sid * (H * W), H * W)])

    plsc.subcore_barrier()

    @pl.loop(0, 4)
    def _batch(bi):
        icps = [pltpu.async_copy(nbr_hbm.at[bi, j, pl.ds(base, CH)],
                                 ibufs[j], sem) for j in range(4)]
        wcps = [pltpu.async_copy(wts_hbm.at[bi, j, pl.ds(base, CH)],
                                 wbufs[j], sem) for j in range(4)]
        for cp in icps:
            cp.wait()
        cps = [pltpu.async_copy(bstage.at[ibufs[j]], gbufs[j], gsem)
               for j in range(4)]
        for cp in wcps:
            cp.wait()
        for cp in cps:
            cp.wait()

        @pl.loop(0, CH // 16)
        def _vec(v):
            s = pl.ds(v * 16, 16)
            acc = ((w0[s] * g0[s] + w1[s] * g1[s])
                   + w2[s] * g2[s]) + w3[s] * g3[s]
            obuf[s] = acc

        pltpu.sync_copy(obuf, out_hbm.at[bi, pl.ds(base, CH)])


def _pool_body(cpl_ref, ipl_ref, alpha_ref, match_ref, src_ref):
    a = alpha_ref[0, 0]
    A = a > 0.0
    bv = None
    for p_ in range(4):
        Cp = cpl_ref[0, p_]
        Ip = ipl_ref[0, p_]
        k = A & (Cp > 0.0)
        v = jnp.where(k, Cp, 0.0)
        im = jnp.where(k, Ip, -1)
        if p_ == 0:
            bv, bi_, anyk = v, im, k
        else:
            upd = v > bv
            bv = jnp.where(upd, v, bv)
            bi_ = jnp.where(upd, im, bi_)
            anyk = anyk | k
    match_ref[0, 0] = anyk.astype(jnp.float32)
    src_ref[0, 0] = bi_


def kernel(alpha, beta, pp, conf, pose_w2c, K, h, w):
    b = pp.shape[0]
    n = pp.shape[2]
    h, w = alpha.shape[-2], alpha.shape[-1]
    f32 = jnp.float32
    i32 = jnp.int32

    pc = jnp.einsum('bij,bjn->bin', pose_w2c, pp[:, :4, :])
    par = jnp.stack([K[:, 0, 0], K[:, 0, 2], K[:, 1, 1], K[:, 1, 2]],
                    axis=1).reshape(b, 1, 4)
    conf3 = conf.reshape(b, 1, n)

    grid = (b, n // PREP_NC)
    sloc3, nbr, wts = pl.pallas_call(
        _prep_body,
        grid=grid,
        in_specs=[
            pl.BlockSpec((1, 4, PREP_NC), lambda bi, j: (bi, 0, j)),
            pl.BlockSpec((1, 1, PREP_NC), lambda bi, j: (bi, 0, j)),
            pl.BlockSpec((1, 1, 4), lambda bi, j: (bi, 0, 0)),
        ],
        out_specs=[
            pl.BlockSpec((1, 1, PREP_NC), lambda bi, j: (bi, 0, j)),
            pl.BlockSpec((1, 4, PREP_NC), lambda bi, j: (bi, 0, j)),
            pl.BlockSpec((1, 4, PREP_NC), lambda bi, j: (bi, 0, j)),
        ],
        out_shape=[
            jax.ShapeDtypeStruct((b, 1, n), i32),
            jax.ShapeDtypeStruct((b, 4, n), i32),
            jax.ShapeDtypeStruct((b, 4, n), f32),
        ],
    )(pc, conf3, par)
    sloc = sloc3.reshape(b, n)

    mesh = plsc.VectorSubcoreMesh(core_axis_name="c", subcore_axis_name="s",
                                  num_cores=2, num_subcores=16)
    sc_cp = pltpu.CompilerParams()
    if "needs_layout_passes" in pltpu.CompilerParams.__dataclass_fields__:
        sc_cp = dataclasses.replace(sc_cp, needs_layout_passes=False)

    sc_scatter = pl.kernel(
        _sc_scatter_body,
        out_type=[jax.ShapeDtypeStruct((b, 4, H, W), f32),
                  jax.ShapeDtypeStruct((b, 4, H, W), i32)],
        mesh=mesh,
        scratch_types=[
            pltpu.VMEM((4, BAND, W), f32),
            pltpu.VMEM((4, BAND, W), i32),
            pltpu.VMEM((SCH,), i32),
            pltpu.VMEM((SCH,), f32),
            pltpu.VMEM((SCH,), i32),
            pltpu.VMEM((SCH,), f32),
            pltpu.SemaphoreType.DMA,
            pltpu.SemaphoreType.DMA,
        ],
        compiler_params=sc_cp,
    )
    cplanes, iplanes = sc_scatter(sloc, conf)

    sc_gather = pl.kernel(
        _sc_gather_body,
        out_type=jax.ShapeDtypeStruct((b, n), f32),
        mesh=mesh,
        scratch_types=(
            [pltpu.VMEM((CH,), i32)] * 4
            + [pltpu.VMEM((CH,), f32)] * 4
            + [pltpu.VMEM((CH,), f32)] * 4
            + [pltpu.VMEM((CH,), f32),
               pltpu.VMEM_SHARED((4 * H * W,), f32),
               pltpu.SemaphoreType.DMA,
               pltpu.SemaphoreType.DMA]
        ),
        compiler_params=sc_cp,
    )
    conf_sampled = sc_gather(beta.reshape(b, h * w), nbr, wts)

    R = 256
    match, src = pl.pallas_call(
        _pool_body,
        grid=(b, H // R),
        in_specs=[
            pl.BlockSpec((1, 4, R, W), lambda bi, rj: (bi, 0, rj, 0)),
            pl.BlockSpec((1, 4, R, W), lambda bi, rj: (bi, 0, rj, 0)),
            pl.BlockSpec((1, 1, R, W), lambda bi, rj: (bi, 0, rj, 0)),
        ],
        out_specs=[
            pl.BlockSpec((1, 1, R, W), lambda bi, rj: (bi, 0, rj, 0)),
            pl.BlockSpec((1, 1, R, W), lambda bi, rj: (bi, 0, rj, 0)),
        ],
        out_shape=[
            jax.ShapeDtypeStruct((b, 1, H, W), f32),
            jax.ShapeDtypeStruct((b, 1, H, W), i32),
        ],
    )(cplanes, iplanes, alpha)

    return (match, src, conf_sampled.reshape(b, 1, n))
